# baseline (speedup 1.0000x reference)
import jax
import jax.numpy as jnp
from jax.experimental import pallas as pl

_N = 96
_REG = 0.05
_N_ITER = 50
_EPS = 1e-9
_THRESH = 0.001


def _emd_body(p_ref, g_ref, out_ref):
    f32 = jnp.float32
    p = p_ref[0, 0, :, :]
    g = g_ref[0, 0, :, :]

    rr = jax.lax.broadcasted_iota(jnp.int32, (_N, _N), 0).astype(f32)
    cc = jax.lax.broadcasted_iota(jnp.int32, (_N, _N), 1).astype(f32)
    eye = (rr == cc).astype(f32)

    pm = p > _THRESH
    gm = g > _THRESH
    pmf = pm.astype(f32)
    gmf = gm.astype(f32)
    n_pre = jnp.sum(pmf)
    n_gt = jnp.sum(gmf)

    pmax = jnp.max(jnp.where(pm, p, -1e30))
    pe = jnp.where(pm, jnp.exp(p - pmax), 0.0)
    A = pe / jnp.sum(pe)
    gmax = jnp.max(jnp.where(gm, g, -1e30))
    ge = jnp.where(gm, jnp.exp(g - gmax), 0.0)
    B = ge / jnp.sum(ge)

    pcmin = jnp.min(jnp.where(pm, cc, 1e9), axis=1, keepdims=True)
    pcmax = jnp.max(jnp.where(pm, cc, -1e9), axis=1, keepdims=True)
    phas = jnp.max(pmf, axis=1, keepdims=True)
    gcmin_c = jnp.min(jnp.where(gm, cc, 1e9), axis=1, keepdims=True)
    gcmax_c = jnp.max(jnp.where(gm, cc, -1e9), axis=1, keepdims=True)
    ghas_c = jnp.max(gmf, axis=1, keepdims=True)

    def t_col_to_row(col):
        return jnp.sum(eye * col, axis=0, keepdims=True)

    gcmin = t_col_to_row(gcmin_c)
    gcmax = t_col_to_row(gcmax_c)
    ghas = t_col_to_row(ghas_c)

    dr2 = (rr - cc) * (rr - cc)
    c00 = (pcmin - gcmin) * (pcmin - gcmin)
    c01 = (pcmin - gcmax) * (pcmin - gcmax)
    c10 = (pcmax - gcmin) * (pcmax - gcmin)
    c11 = (pcmax - gcmax) * (pcmax - gcmax)
    cmax = jnp.maximum(jnp.maximum(c00, c01), jnp.maximum(c10, c11))
    valid = (phas * ghas) > 0.0
    m_max = jnp.max(jnp.where(valid, dr2 + cmax, 0.0))

    alpha = 1.0 / (m_max * _REG)
    d2 = dr2
    G1 = jnp.exp(-alpha * d2)
    G2 = d2 * G1

    dot = lambda x, y: jax.lax.dot(
        x, y, precision=jax.lax.Precision.DEFAULT,
        preferred_element_type=f32)

    H = _N // 2

    def conv(X):
        W0 = dot(X[:H, :], G1)
        W1 = dot(X[H:, :], G1)
        return dot(G1[:, :H], W0) + dot(G1[:, H:], W1)

    V = gmf
    U = jnp.zeros((_N, _N), f32)
    for _ in range(_N_ITER):
        U = A / (conv(V) + _EPS)
        V = B / (conv(U) + _EPS)

    P = dot(G2, dot(V, G1)) + dot(G1, dot(V, G2))
    loss = jnp.sum(U * P)
    out_ref[:, :] = jnp.reshape(loss / n_gt / n_pre, (1, 1))


def kernel(pred, gt):
    spec = pl.BlockSpec((1, 1, _N, _N), lambda i: (0, 0, 0, 0))
    out = pl.pallas_call(
        _emd_body,
        grid=(1,),
        in_specs=[spec, spec],
        out_specs=pl.BlockSpec((1, 1), lambda i: (0, 0)),
        out_shape=jax.ShapeDtypeStruct((1, 1), jnp.float32),
    )(pred, gt)
    return out[0, 0]


# speedup vs baseline: 1.0769x; 1.0769x over previous
import jax
import jax.numpy as jnp
from jax.experimental import pallas as pl

_N = 96
_REG = 0.05
_N_ITER = 50
_EPS = 1e-9
_THRESH = 0.001


def _emd_body(p_ref, g_ref, out_ref):
    f32 = jnp.float32
    p = p_ref[0, 0, :, :]
    g = g_ref[0, 0, :, :]

    rr = jax.lax.broadcasted_iota(jnp.int32, (_N, _N), 0).astype(f32)
    cc = jax.lax.broadcasted_iota(jnp.int32, (_N, _N), 1).astype(f32)
    eye = (rr == cc).astype(f32)

    pm = p > _THRESH
    gm = g > _THRESH
    pmf = pm.astype(f32)
    gmf = gm.astype(f32)
    n_pre = jnp.sum(pmf)
    n_gt = jnp.sum(gmf)

    pmax = jnp.max(jnp.where(pm, p, -1e30))
    pe = jnp.where(pm, jnp.exp(p - pmax), 0.0)
    A = pe / jnp.sum(pe)
    gmax = jnp.max(jnp.where(gm, g, -1e30))
    ge = jnp.where(gm, jnp.exp(g - gmax), 0.0)
    B = ge / jnp.sum(ge)

    pcmin = jnp.min(jnp.where(pm, cc, 1e9), axis=1, keepdims=True)
    pcmax = jnp.max(jnp.where(pm, cc, -1e9), axis=1, keepdims=True)
    phas = jnp.max(pmf, axis=1, keepdims=True)
    gcmin_c = jnp.min(jnp.where(gm, cc, 1e9), axis=1, keepdims=True)
    gcmax_c = jnp.max(jnp.where(gm, cc, -1e9), axis=1, keepdims=True)
    ghas_c = jnp.max(gmf, axis=1, keepdims=True)

    def t_col_to_row(col):
        return jnp.sum(eye * col, axis=0, keepdims=True)

    gcmin = t_col_to_row(gcmin_c)
    gcmax = t_col_to_row(gcmax_c)
    ghas = t_col_to_row(ghas_c)

    dr2 = (rr - cc) * (rr - cc)
    c00 = (pcmin - gcmin) * (pcmin - gcmin)
    c01 = (pcmin - gcmax) * (pcmin - gcmax)
    c10 = (pcmax - gcmin) * (pcmax - gcmin)
    c11 = (pcmax - gcmax) * (pcmax - gcmax)
    cmax = jnp.maximum(jnp.maximum(c00, c01), jnp.maximum(c10, c11))
    valid = (phas * ghas) > 0.0
    m_max = jnp.max(jnp.where(valid, dr2 + cmax, 0.0))

    alpha = 1.0 / (m_max * _REG)
    d2 = dr2
    G1 = jnp.exp(-alpha * d2)
    G2 = d2 * G1

    dot = lambda x, y: jax.lax.dot(
        x, y, precision=jax.lax.Precision.DEFAULT,
        preferred_element_type=f32)

    def conv(X):
        return dot(dot(G1, X), G1)

    V = gmf
    U = jnp.zeros((_N, _N), f32)
    for _ in range(_N_ITER):
        U = A / (conv(V) + _EPS)
        V = B / (conv(U) + _EPS)

    P = dot(G2, dot(V, G1)) + dot(G1, dot(V, G2))
    loss = jnp.sum(U * P)
    out_ref[:, :] = jnp.reshape(loss / n_gt / n_pre, (1, 1))


def kernel(pred, gt):
    spec = pl.BlockSpec((1, 1, _N, _N), lambda i: (0, 0, 0, 0))
    out = pl.pallas_call(
        _emd_body,
        grid=(1,),
        in_specs=[spec, spec],
        out_specs=pl.BlockSpec((1, 1), lambda i: (0, 0)),
        out_shape=jax.ShapeDtypeStruct((1, 1), jnp.float32),
    )(pred, gt)
    return out[0, 0]
